# initial kernel scaffold (unmeasured)
import jax
import jax.numpy as jnp
from jax import lax
from jax.experimental import pallas as pl
from jax.experimental.pallas import tpu as pltpu

N_DEV = 32


def kernel(A, B):
    m_per, k = A.shape
    k2, n = B.shape
    assert k == k2

    def body(a_ref, b_ref, out_ref, comm_ref, stage_ref, send_sems, recv_sems,
             copy_sem, credit_sem):
        my = lax.axis_index("i")
        left = lax.rem(my + (N_DEV - 1), N_DEV)
        right = lax.rem(my + 1, N_DEV)

        barrier_sem = pltpu.get_barrier_semaphore()
        for nbr in (left, right):
            pl.semaphore_signal(
                barrier_sem, inc=1,
                device_id=(nbr,), device_id_type=pl.DeviceIdType.MESH,
            )
        pl.semaphore_wait(barrier_sem, 2)

        pl.semaphore_signal(
            credit_sem, inc=1,
            device_id=(left,), device_id_type=pl.DeviceIdType.MESH,
        )

        b_bf = b_ref[:].astype(jnp.bfloat16)

        comm_ref[0] = a_ref[:].astype(jnp.bfloat16)
        stage_ref[:] = jnp.dot(
            comm_ref[0], b_bf, preferred_element_type=jnp.float32
        ).astype(jnp.bfloat16)
        copy = pltpu.make_async_copy(
            stage_ref, out_ref.at[pl.ds(my * m_per, m_per), :], copy_sem
        )
        copy.start()
        copy.wait()

        def hop(h, carry):
            s = lax.rem(h, 2)
            r = lax.rem(h + 1, 2)
            pl.semaphore_wait(credit_sem, 1)
            rdma = pltpu.make_async_remote_copy(
                src_ref=comm_ref.at[s],
                dst_ref=comm_ref.at[r],
                send_sem=send_sems.at[s],
                recv_sem=recv_sems.at[r],
                device_id=(right,),
                device_id_type=pl.DeviceIdType.MESH,
            )
            rdma.start()
            rdma.wait()
            @pl.when(h < N_DEV - 2)
            def _():
                pl.semaphore_signal(
                    credit_sem, inc=1,
                    device_id=(left,), device_id_type=pl.DeviceIdType.MESH,
                )

            origin = lax.rem(my + (N_DEV - 1 - h), N_DEV)
            stage_ref[:] = jnp.dot(
                comm_ref[r], b_bf, preferred_element_type=jnp.float32
            ).astype(jnp.bfloat16)
            copy = pltpu.make_async_copy(
                stage_ref, out_ref.at[pl.ds(origin * m_per, m_per), :], copy_sem
            )
            copy.start()
            copy.wait()
            return carry

        lax.fori_loop(0, N_DEV - 1, hop, 0)

    return pl.pallas_call(
        body,
        out_shape=jax.ShapeDtypeStruct((N_DEV * m_per, n), jnp.bfloat16),
        in_specs=[
            pl.BlockSpec(memory_space=pltpu.VMEM),
            pl.BlockSpec(memory_space=pltpu.VMEM),
        ],
        out_specs=pl.BlockSpec(memory_space=pltpu.ANY),
        scratch_shapes=[
            pltpu.VMEM((2, m_per, k), jnp.bfloat16),
            pltpu.VMEM((m_per, n), jnp.bfloat16),
            pltpu.SemaphoreType.DMA((2,)),
            pltpu.SemaphoreType.DMA((2,)),
            pltpu.SemaphoreType.DMA,
            pltpu.SemaphoreType.REGULAR,
        ],
        compiler_params=pltpu.CompilerParams(collective_id=0),
    )(A, B)


# baseline (device time: 1151268 ns/iter reference)
import jax
import jax.numpy as jnp
from jax import lax
from jax.experimental import pallas as pl
from jax.experimental.pallas import tpu as pltpu

N_DEV = 32


def kernel(A, B):
    m_per, k = A.shape
    k2, n = B.shape
    assert k == k2

    def body(a_ref, b_ref, out_ref, comm_ref, stage_ref, send_sems, recv_sems,
             copy_sem, credit_sem):
        my = lax.axis_index("i")
        left = lax.rem(my + (N_DEV - 1), N_DEV)
        right = lax.rem(my + 1, N_DEV)

        barrier_sem = pltpu.get_barrier_semaphore()
        for nbr in (left, right):
            pl.semaphore_signal(
                barrier_sem, inc=1,
                device_id=(nbr,), device_id_type=pl.DeviceIdType.MESH,
            )
        pl.semaphore_wait(barrier_sem, 2)

        pl.semaphore_signal(
            credit_sem, inc=1,
            device_id=(left,), device_id_type=pl.DeviceIdType.MESH,
        )

        b_bf = b_ref[:].astype(jnp.bfloat16)

        comm_ref[0] = a_ref[:].astype(jnp.bfloat16)
        stage_ref[:] = jnp.dot(
            comm_ref[0], b_bf, preferred_element_type=jnp.float32
        ).astype(jnp.bfloat16)
        copy = pltpu.make_async_copy(
            stage_ref, out_ref.at[pl.ds(my * m_per, m_per), :], copy_sem
        )
        copy.start()
        copy.wait()

        def hop(h, carry):
            s = lax.rem(h, 2)
            r = lax.rem(h + 1, 2)
            pl.semaphore_wait(credit_sem, 1)
            rdma = pltpu.make_async_remote_copy(
                src_ref=comm_ref.at[s],
                dst_ref=comm_ref.at[r],
                send_sem=send_sems.at[s],
                recv_sem=recv_sems.at[r],
                device_id=(right,),
                device_id_type=pl.DeviceIdType.MESH,
            )
            rdma.start()
            rdma.wait()
            @pl.when(h < N_DEV - 2)
            def _():
                pl.semaphore_signal(
                    credit_sem, inc=1,
                    device_id=(left,), device_id_type=pl.DeviceIdType.MESH,
                )

            origin = lax.rem(my + (N_DEV - 1 - h), N_DEV)
            stage_ref[:] = jnp.dot(
                comm_ref[r], b_bf, preferred_element_type=jnp.float32
            ).astype(jnp.bfloat16)
            copy = pltpu.make_async_copy(
                stage_ref, out_ref.at[pl.ds(origin * m_per, m_per), :], copy_sem
            )
            copy.start()
            copy.wait()
            return carry

        lax.fori_loop(0, N_DEV - 1, hop, 0)

    return pl.pallas_call(
        body,
        out_shape=jax.ShapeDtypeStruct((N_DEV * m_per, n), jnp.bfloat16),
        in_specs=[
            pl.BlockSpec(memory_space=pltpu.MemorySpace.VMEM),
            pl.BlockSpec(memory_space=pltpu.MemorySpace.VMEM),
        ],
        out_specs=pl.BlockSpec(memory_space=pl.ANY),
        scratch_shapes=[
            pltpu.MemorySpace.VMEM((2, m_per, k), jnp.bfloat16),
            pltpu.MemorySpace.VMEM((m_per, n), jnp.bfloat16),
            pltpu.SemaphoreType.DMA((2,)),
            pltpu.SemaphoreType.DMA((2,)),
            pltpu.SemaphoreType.DMA,
            pltpu.SemaphoreType.REGULAR,
        ],
        compiler_params=pltpu.CompilerParams(collective_id=0),
    )(A, B)


# device time: 550485 ns/iter; 2.0914x vs baseline; 2.0914x over previous
import numpy as np

import jax
import jax.numpy as jnp
from jax import lax
from jax.experimental import pallas as pl
from jax.experimental.pallas import tpu as pltpu

N_DEV = 32
NR = 16
NL = 15


def _ring_tables():
    plane = [(0, 0), (1, 0), (1, 1), (0, 1), (0, 2), (1, 2), (1, 3), (0, 3)]
    logical_of_coord = {}
    for z in range(4):
        for k, (x, y) in enumerate(plane):
            logical_of_coord[(x, y, z)] = z * 8 + k
    cyc = []
    for y in range(4):
        zs = range(4) if y % 2 == 0 else range(3, -1, -1)
        cyc += [(0, y, z) for z in zs]
    for y in range(3, -1, -1):
        zs = range(4) if (3 - y) % 2 == 0 else range(3, -1, -1)
        cyc += [(1, y, z) for z in zs]
    for i in range(32):
        a, b = cyc[i], cyc[(i + 1) % 32]
        assert sum(abs(p - q) for p, q in zip(a, b)) == 1, (i, a, b)
    perm = np.array([logical_of_coord[c] for c in cyc], dtype=np.int32)
    inv = np.empty(32, dtype=np.int32)
    inv[perm] = np.arange(32, dtype=np.int32)
    return perm, inv


_PERM, _INV = _ring_tables()


def kernel(A, B):
    m_per, k = A.shape
    k2, n = B.shape
    assert k == k2

    perm = jnp.asarray(_PERM)
    my = lax.axis_index("i")
    r = jnp.asarray(_INV)[my]
    left = perm[(r + N_DEV - 1) % N_DEV]
    right = perm[(r + 1) % N_DEV]
    origin_r = perm[(r + N_DEV - 1 - jnp.arange(NR)) % N_DEV]
    origin_l = perm[(r + 1 + jnp.arange(NL)) % N_DEV]
    meta = jnp.concatenate(
        [left[None], right[None], origin_r, origin_l]
    ).astype(jnp.int32)

    def body(a_ref, b_ref, meta_ref, out_ref, comm_r, comm_l, stage_r, stage_l,
             send_sems_r, recv_sems_r, send_sems_l, recv_sems_l,
             copy_sem_r, copy_sem_l, credit_r, credit_l):
        my = lax.axis_index("i")
        left = meta_ref[0]
        right = meta_ref[1]

        def desc_r(src_slot, dst_slot, sem_slot, rsem_slot, target):
            return pltpu.make_async_remote_copy(
                src_ref=comm_r.at[src_slot],
                dst_ref=comm_r.at[dst_slot],
                send_sem=send_sems_r.at[sem_slot],
                recv_sem=recv_sems_r.at[rsem_slot],
                device_id=(target,),
                device_id_type=pl.DeviceIdType.MESH,
            )

        def desc_l(src_slot, dst_slot, sem_slot, rsem_slot, target):
            return pltpu.make_async_remote_copy(
                src_ref=comm_l.at[src_slot],
                dst_ref=comm_l.at[dst_slot],
                send_sem=send_sems_l.at[sem_slot],
                recv_sem=recv_sems_l.at[rsem_slot],
                device_id=(target,),
                device_id_type=pl.DeviceIdType.MESH,
            )

        def grant(sem, target):
            pl.semaphore_signal(
                sem, inc=1, device_id=(target,),
                device_id_type=pl.DeviceIdType.MESH,
            )

        barrier_sem = pltpu.get_barrier_semaphore()
        for nbr in (left, right):
            pl.semaphore_signal(
                barrier_sem, inc=1,
                device_id=(nbr,), device_id_type=pl.DeviceIdType.MESH,
            )
        pl.semaphore_wait(barrier_sem, 2)

        grant(credit_r, left)
        grant(credit_l, right)

        b_bf = b_ref[:].astype(jnp.bfloat16)
        a_bf = a_ref[:].astype(jnp.bfloat16)
        comm_r[0] = a_bf
        comm_l[0] = a_bf

        pl.semaphore_wait(credit_r, 1)
        send_r0 = desc_r(0, 1, 0, 1, right)
        send_r0.start()
        pl.semaphore_wait(credit_l, 1)
        send_l0 = desc_l(0, 1, 0, 1, left)
        send_l0.start()

        stage_r[:] = jnp.dot(
            a_bf, b_bf, preferred_element_type=jnp.float32
        ).astype(jnp.bfloat16)
        copy = pltpu.make_async_copy(
            stage_r, out_ref.at[pl.ds(my * m_per, m_per), :], copy_sem_r
        )
        copy.start()
        copy.wait()

        send_r0.wait_send()
        grant(credit_r, left)
        send_l0.wait_send()
        grant(credit_l, right)

        def hop(h, carry):
            s = lax.rem(h, 2)
            d = lax.rem(h + 1, 2)

            desc_r(d, d, d, d, left).wait_recv()

            @pl.when(h < NL)
            def _():
                desc_l(d, d, d, d, right).wait_recv()

            @pl.when(h + 1 < NR)
            def _():
                pl.semaphore_wait(credit_r, 1)
                desc_r(d, s, d, s, right).start()

            @pl.when(h + 1 < NL)
            def _():
                pl.semaphore_wait(credit_l, 1)
                desc_l(d, s, d, s, left).start()

            stage_r[:] = jnp.dot(
                comm_r[d], b_bf, preferred_element_type=jnp.float32
            ).astype(jnp.bfloat16)
            cr = pltpu.make_async_copy(
                stage_r,
                out_ref.at[pl.ds(meta_ref[2 + h] * m_per, m_per), :],
                copy_sem_r,
            )
            cr.start()

            @pl.when(h < NL)
            def _():
                stage_l[:] = jnp.dot(
                    comm_l[d], b_bf, preferred_element_type=jnp.float32
                ).astype(jnp.bfloat16)
                cl = pltpu.make_async_copy(
                    stage_l,
                    out_ref.at[pl.ds(meta_ref[2 + NR + h] * m_per, m_per), :],
                    copy_sem_l,
                )
                cl.start()
                cl.wait()

            cr.wait()

            @pl.when(h + 1 < NR)
            def _():
                desc_r(d, s, d, s, right).wait_send()

                @pl.when(h + 1 < NR - 1)
                def _():
                    grant(credit_r, left)

            @pl.when(h + 1 < NL)
            def _():
                desc_l(d, s, d, s, left).wait_send()

                @pl.when(h + 1 < NL - 1)
                def _():
                    grant(credit_l, right)

            return carry

        lax.fori_loop(0, NR, hop, 0)

    out = pl.pallas_call(
        body,
        out_shape=jax.ShapeDtypeStruct((N_DEV * m_per, n), jnp.bfloat16),
        in_specs=[
            pl.BlockSpec(memory_space=pltpu.MemorySpace.VMEM),
            pl.BlockSpec(memory_space=pltpu.MemorySpace.VMEM),
            pl.BlockSpec(memory_space=pltpu.MemorySpace.SMEM),
        ],
        out_specs=pl.BlockSpec(memory_space=pl.ANY),
        scratch_shapes=[
            pltpu.MemorySpace.VMEM((2, m_per, k), jnp.bfloat16),
            pltpu.MemorySpace.VMEM((2, m_per, k), jnp.bfloat16),
            pltpu.MemorySpace.VMEM((m_per, n), jnp.bfloat16),
            pltpu.MemorySpace.VMEM((m_per, n), jnp.bfloat16),
            pltpu.SemaphoreType.DMA((2,)),
            pltpu.SemaphoreType.DMA((2,)),
            pltpu.SemaphoreType.DMA((2,)),
            pltpu.SemaphoreType.DMA((2,)),
            pltpu.SemaphoreType.DMA,
            pltpu.SemaphoreType.DMA,
            pltpu.SemaphoreType.REGULAR,
            pltpu.SemaphoreType.REGULAR,
        ],
        compiler_params=pltpu.CompilerParams(collective_id=0),
    )(A, B, meta)
    return out
